# Initial kernel scaffold; baseline (speedup 1.0000x reference)
import jax
import jax.numpy as jnp
from jax.experimental import pallas as pl


def kernel(x, emb, W, b):
    raise NotImplementedError("write your pallas kernel here")


# speedup vs baseline: 49.6180x; 49.6180x over previous
import functools

import jax
import jax.numpy as jnp
from jax.experimental import pallas as pl


def _body(x_ref, emb_ref, w_ref, b_ref, o0, o1, o2, o3, o4):
    t = jax.lax.dot_general(
        emb_ref[...], w_ref[...],
        dimension_numbers=(((1,), (1,)), ((), ())),
        preferred_element_type=jnp.float32,
    ) + b_ref[...]
    xb = x_ref[...]
    outs = [o0, o1, o2, o3, o4]
    accs = [jnp.zeros(xb.shape, jnp.float32) for _ in range(5)]
    for v in range(10):
        m = (xb == v).astype(jnp.float32)
        for c in range(5):
            tv = jax.lax.slice(t, (v, c), (v + 1, c + 1))
            accs[c] = accs[c] + m * tv
    for c in range(5):
        outs[c][...] = accs[c]


@functools.partial(jax.jit, static_argnames=("interpret",))
def _run(x, emb, W, b, interpret=False):
    B, L = x.shape
    bm = 512
    grid = (B // bm,)
    outs = pl.pallas_call(
        _body,
        grid=grid,
        in_specs=[
            pl.BlockSpec((bm, L), lambda i: (i, 0)),
            pl.BlockSpec((10, 20), lambda i: (0, 0)),
            pl.BlockSpec((5, 20), lambda i: (0, 0)),
            pl.BlockSpec((1, 5), lambda i: (0, 0)),
        ],
        out_specs=[pl.BlockSpec((bm, L), lambda i: (i, 0))] * 5,
        out_shape=[jax.ShapeDtypeStruct((B, L), jnp.float32)] * 5,
        interpret=interpret,
    )(x, emb, W, b)
    return jnp.stack(outs, axis=-1)


def kernel(x, emb, W, b):
    return _run(x, emb, W, b)
